# baseline (speedup 1.0000x reference)
import jax
import jax.numpy as jnp
from jax.experimental import pallas as pl
from jax.experimental.pallas import tpu as pltpu

EPS = 1e-5
BF = jnp.bfloat16
F32 = jnp.float32

_CompilerParams = getattr(pltpu, "CompilerParams", None) or pltpu.TPUCompilerParams

_TA = (((0,), (0,)), ((), ()))


def _ln_rows(h, g, be):
    m = jnp.mean(h, axis=-1, keepdims=True)
    c = h - m
    v = jnp.mean(c * c, axis=-1, keepdims=True)
    return c * jax.lax.rsqrt(v + EPS) * g + be


def _pad_rows(v, n):
    return jnp.concatenate(
        [v, jnp.zeros((n - v.shape[0], v.shape[1]), v.dtype)], axis=0)


def _weights_kernel(aw_ref, w1_ref, mem_ref, noi_ref, res_ref,
                    mw1, mw2, nw1, nw2, rw1, rw2,
                    cpb_ref, tmb_ref, msb_ref, auxp_ref,
                    w1m_ref, proj_ref, bias_ref,
                    acc_ref, stash_ref):
    i = pl.program_id(0)
    D = w1_ref.shape[0]
    av = aw_ref[...]
    ex = jnp.exp(av - jnp.max(av))
    wv = ex / jnp.sum(ex)
    wi = jnp.sum(jnp.where(jax.lax.iota(jnp.int32, 6) == i, wv, 0.0),
                 keepdims=True).reshape(1, 1)
    sblk = w1_ref[...] * wi
    sb16 = sblk.astype(BF)

    @pl.when((i == 0) | (i == 1) | (i == 5))
    def _():
        w1m_ref[...] = sb16

    @pl.when(i == 0)
    def _():
        proj_ref[0:8] = jnp.dot(_pad_rows(cpb_ref[...], 8).astype(BF), sb16,
                                preferred_element_type=F32)

    @pl.when(i == 1)
    def _():
        proj_ref[8:16] = jnp.dot(_pad_rows(tmb_ref[...], 8).astype(BF), sb16,
                                 preferred_element_type=F32)

    @pl.when(i == 2)
    def _():
        acc_ref[...] = sblk
        stash_ref[0:D] = sb16

    @pl.when(i == 3)
    def _():
        acc_ref[...] = acc_ref[...] + sblk
        stash_ref[D:2 * D] = sb16

    @pl.when(i == 4)
    def _():
        acc_ref[...] = acc_ref[...] + sblk
        stash_ref[2 * D:3 * D] = sb16
        w1m_ref[...] = acc_ref[...].astype(BF)

    @pl.when(i == 5)
    def _():
        proj_ref[16:24] = jnp.dot(_pad_rows(msb_ref[...], 8).astype(BF), sb16,
                                  preferred_element_type=F32)

        def mlp_ln(st_ref, w1r, w2r, r0):
            hpre = jnp.dot(st_ref[...].astype(BF), w1r[...].astype(BF),
                           preferred_element_type=F32) + auxp_ref[r0:r0 + 1]
            hmid = hpre * jax.nn.sigmoid(hpre)
            hv = jnp.dot(hmid.astype(BF), w2r[...].astype(BF),
                         preferred_element_type=F32) + auxp_ref[r0 + 1:r0 + 2]
            return _ln_rows(hv, auxp_ref[r0 + 2:r0 + 3],
                            auxp_ref[r0 + 3:r0 + 4])

        mv = mlp_ln(mem_ref, mw1, mw2, 0)
        nv = mlp_ln(noi_ref, nw1, nw2, 4)
        rv = mlp_ln(res_ref, rw1, rw2, 8)
        catv = jnp.concatenate(
            [mv.astype(BF), nv.astype(BF), rv.astype(BF)], axis=-1)
        bias_ref[...] = (jnp.dot(catv, stash_ref[...],
                                 preferred_element_type=F32)
                         + auxp_ref[12:13]).reshape(bias_ref.shape)


def _tables_kernel(pwg_ref, pwb_ref, w2_ref, cpg_ref, tmg_ref, msg_ref,
                   pwcat_ref, w2b_ref, small_ref):
    D = pwg_ref.shape[1]
    npad = pwcat_ref.shape[0] - pwg_ref.shape[0]
    zp = jnp.zeros((npad, D), F32)
    pwcat_ref[:, 0:D] = jnp.concatenate(
        [pwg_ref[...] - 1.0, zp], axis=0).astype(BF)
    pwcat_ref[:, D:2 * D] = jnp.concatenate(
        [pwb_ref[...], zp], axis=0).astype(BF)
    w2b_ref[...] = w2_ref[...].astype(BF)
    small_ref[...] = jnp.concatenate(
        [_pad_rows(cpg_ref[...] - 1.0, 8), _pad_rows(tmg_ref[...] - 1.0, 8),
         _pad_rows(msg_ref[...] - 1.0, 8)], axis=0).astype(BF)


def _main_kernel(x_ref, ids_ref, pw_ref, small_ref, w1m_ref, proj_ref, w2_ref,
                 bias_ref, auxp_ref, o_ref):
    D = x_ref.shape[-1]
    tile = x_ref.shape[1]
    xb = x_ref[0]
    ids = ids_ref[0, 0]
    pid = ids[0:1, :]
    cid = ids[1:2, :]
    tid = ids[2:3, :]
    sid = ids[3:4, :]

    mu = jnp.mean(xb, axis=-1, keepdims=True)
    sxx = jnp.mean(xb * xb, axis=-1, keepdims=True)
    xhat = (xb - mu) * jax.lax.rsqrt(sxx - mu * mu + EPS)

    npw = pw_ref.shape[0]
    iota_pw = jax.lax.broadcasted_iota(jnp.int32, (npw, tile), 0)
    ohT_p = jnp.where(iota_pw == pid, 1.0, 0.0).astype(BF)
    pwgb = jax.lax.dot_general(ohT_p, pw_ref[...], _TA,
                               preferred_element_type=F32)
    h1 = xhat * (pwgb[:, :D] + 1.0) + pwgb[:, D:]

    mu1 = jnp.mean(h1, axis=-1, keepdims=True)
    s11 = jnp.mean(h1 * h1, axis=-1, keepdims=True)
    h1h = (h1 - mu1) * jax.lax.rsqrt(s11 - mu1 * mu1 + EPS)

    iota8 = jax.lax.broadcasted_iota(jnp.int32, (8, tile), 0)
    ohT_c = jnp.where(iota8 == cid, 1.0, 0.0).astype(BF)
    ohT_t = jnp.where(iota8 == tid, 1.0, 0.0).astype(BF)
    ohT_s = jnp.where(iota8 == sid, 1.0, 0.0).astype(BF)
    cg1 = jax.lax.dot_general(ohT_c, small_ref[0:8], _TA,
                              preferred_element_type=F32)
    tg1 = jax.lax.dot_general(ohT_t, small_ref[8:16], _TA,
                              preferred_element_type=F32)
    sg1 = jax.lax.dot_general(ohT_s, small_ref[16:24], _TA,
                              preferred_element_type=F32)

    h = h1h * (cg1 + 1.0)
    t = xhat * (tg1 + 1.0)
    s = xhat * (sg1 + 1.0)

    iota24 = jax.lax.broadcasted_iota(jnp.int32, (24, tile), 0)
    m24 = (iota24 == cid) | (iota24 - 8 == tid) | (iota24 - 16 == sid)
    oh24T = jnp.where(m24, 1.0, 0.0)
    badd = jax.lax.dot_general(oh24T, proj_ref[...], _TA,
                               preferred_element_type=F32)

    big = jnp.concatenate(
        [h.astype(BF), t.astype(BF), s.astype(BF), xb.astype(BF)], axis=-1)
    pre = (jnp.dot(big, w1m_ref[...], preferred_element_type=F32)
           + badd + bias_ref[0])
    hid = pre * jax.nn.sigmoid(pre)
    h2 = jnp.dot(hid.astype(BF), w2_ref[...],
                 preferred_element_type=F32) + auxp_ref[13:14]

    mu2 = jnp.mean(h2, axis=-1, keepdims=True)
    s22 = jnp.mean(h2 * h2, axis=-1, keepdims=True)
    o_ref[0] = ((h2 - mu2) * jax.lax.rsqrt(s22 - mu2 * mu2 + EPS)
                ) * auxp_ref[14:15] + auxp_ref[15:16]


def kernel(x, pathway_ids, compartment_ids, time_steps, scale_type,
           memory_state, noise_state, resource_state,
           pw_g, pw_b, cp_g, cp_b, tm_g, tm_b, ms_g, ms_b,
           mem_W1, mem_b1, mem_W2, mem_b2, mem_g, mem_be,
           noi_W1, noi_b1, noi_W2, noi_b2, noi_g, noi_be,
           res_W1, res_b1, res_W2, res_b2, res_g, res_be,
           int_W1, int_b1, int_W2, int_b2, int_g, int_be, aw):
    B, S, D = x.shape
    TILE = 256
    NB = S // TILE

    npw = pw_g.shape[0]
    npw_pad = ((npw + 127) // 128) * 128

    auxp = jnp.stack([mem_b1, mem_b2, mem_g, mem_be,
                      noi_b1, noi_b2, noi_g, noi_be,
                      res_b1, res_b2, res_g, res_be,
                      int_b1, int_b2, int_g, int_be], axis=0)

    def _w1m_row(i):
        return jnp.where(i < 2, i, jnp.where(i == 5, 2, 3))

    w1m, proj, bias_rows = pl.pallas_call(
        _weights_kernel,
        out_shape=(jax.ShapeDtypeStruct((4 * D, D), BF),
                   jax.ShapeDtypeStruct((24, D), F32),
                   jax.ShapeDtypeStruct((B, 1, D), F32)),
        grid=(6,),
        in_specs=[
            pl.BlockSpec((6,), lambda i: (0,)),
            pl.BlockSpec((D, D), lambda i: (i, 0)),
            pl.BlockSpec((B, 512), lambda i: (0, 0)),
            pl.BlockSpec((B, 64), lambda i: (0, 0)),
            pl.BlockSpec((B, 32), lambda i: (0, 0)),
            pl.BlockSpec((512, D), lambda i: (0, 0)),
            pl.BlockSpec((D, D), lambda i: (0, 0)),
            pl.BlockSpec((64, D), lambda i: (0, 0)),
            pl.BlockSpec((D, D), lambda i: (0, 0)),
            pl.BlockSpec((32, D), lambda i: (0, 0)),
            pl.BlockSpec((D, D), lambda i: (0, 0)),
            pl.BlockSpec((5, D), lambda i: (0, 0)),
            pl.BlockSpec((5, D), lambda i: (0, 0)),
            pl.BlockSpec((3, D), lambda i: (0, 0)),
            pl.BlockSpec((16, D), lambda i: (0, 0)),
        ],
        out_specs=(pl.BlockSpec((D, D), lambda i: (_w1m_row(i), 0)),
                   pl.BlockSpec((24, D), lambda i: (0, 0)),
                   pl.BlockSpec((B, 1, D), lambda i: (0, 0, 0))),
        scratch_shapes=[pltpu.VMEM((D, D), F32),
                        pltpu.VMEM((3 * D, D), BF)],
        compiler_params=_CompilerParams(
            dimension_semantics=("arbitrary",),
            vmem_limit_bytes=56 * 1024 * 1024,
        ),
        name="weights_prep",
    )(aw, int_W1, memory_state, noise_state, resource_state,
      mem_W1, mem_W2, noi_W1, noi_W2, res_W1, res_W2,
      cp_b, tm_b, ms_b, auxp)

    pw_cat, w2b, small = pl.pallas_call(
        _tables_kernel,
        out_shape=(jax.ShapeDtypeStruct((npw_pad, 2 * D), BF),
                   jax.ShapeDtypeStruct((D, D), BF),
                   jax.ShapeDtypeStruct((24, D), BF)),
        name="tables_prep",
    )(pw_g, pw_b, int_W2, cp_g, tm_g, ms_g)

    ids_all = jnp.stack(
        [pathway_ids.reshape(B, NB, TILE), compartment_ids.reshape(B, NB, TILE),
         time_steps.reshape(B, NB, TILE), scale_type.reshape(B, NB, TILE)],
        axis=2).astype(jnp.int32)

    out = pl.pallas_call(
        _main_kernel,
        out_shape=jax.ShapeDtypeStruct((B, S, D), F32),
        grid=(B, NB),
        in_specs=[
            pl.BlockSpec((1, TILE, D), lambda b, j: (b, j, 0)),
            pl.BlockSpec((1, 1, 4, TILE), lambda b, j: (b, j, 0, 0)),
            pl.BlockSpec((npw_pad, 2 * D), lambda b, j: (0, 0)),
            pl.BlockSpec((24, D), lambda b, j: (0, 0)),
            pl.BlockSpec((4 * D, D), lambda b, j: (0, 0)),
            pl.BlockSpec((24, D), lambda b, j: (0, 0)),
            pl.BlockSpec((D, D), lambda b, j: (0, 0)),
            pl.BlockSpec((1, 1, D), lambda b, j: (b, 0, 0)),
            pl.BlockSpec((16, D), lambda b, j: (0, 0)),
        ],
        out_specs=pl.BlockSpec((1, TILE, D), lambda b, j: (b, j, 0)),
        compiler_params=_CompilerParams(
            dimension_semantics=("parallel", "arbitrary"),
            vmem_limit_bytes=56 * 1024 * 1024,
        ),
        name="comprehensive_norm",
    )(x, ids_all, pw_cat, small, w1m, proj, w2b, bias_rows, auxp)
    return out


# speedup vs baseline: 1.0431x; 1.0431x over previous
import jax
import jax.numpy as jnp
from jax.experimental import pallas as pl
from jax.experimental.pallas import tpu as pltpu

EPS = 1e-5
BF = jnp.bfloat16
F32 = jnp.float32

_CompilerParams = getattr(pltpu, "CompilerParams", None) or pltpu.TPUCompilerParams

_TA = (((0,), (0,)), ((), ()))


def _ln_rows(h, g, be):
    m = jnp.mean(h, axis=-1, keepdims=True)
    c = h - m
    v = jnp.mean(c * c, axis=-1, keepdims=True)
    return c * jax.lax.rsqrt(v + EPS) * g + be


def _pad_rows(v, n):
    return jnp.concatenate(
        [v, jnp.zeros((n - v.shape[0], v.shape[1]), v.dtype)], axis=0)


def _weights_kernel(aw_ref, w1_ref, mem_ref, noi_ref, res_ref,
                    mw1, mw2, nw1, nw2, rw1, rw2,
                    cpb_ref, tmb_ref, msb_ref, auxp_ref,
                    w1m_ref, proj_ref, bias_ref,
                    acc_ref, stash_ref):
    i = pl.program_id(0)
    D = w1_ref.shape[0]
    av = aw_ref[...]
    ex = jnp.exp(av - jnp.max(av))
    wv = ex / jnp.sum(ex)
    wi = jnp.sum(jnp.where(jax.lax.iota(jnp.int32, 6) == i, wv, 0.0),
                 keepdims=True).reshape(1, 1)
    sblk = w1_ref[...] * wi
    sb16 = sblk.astype(BF)

    @pl.when((i == 0) | (i == 1) | (i == 5))
    def _():
        w1m_ref[...] = sb16

    @pl.when(i == 0)
    def _():
        proj_ref[0:8] = jnp.dot(_pad_rows(cpb_ref[...], 8).astype(BF), sb16,
                                preferred_element_type=F32)

    @pl.when(i == 1)
    def _():
        proj_ref[8:16] = jnp.dot(_pad_rows(tmb_ref[...], 8).astype(BF), sb16,
                                 preferred_element_type=F32)

    @pl.when(i == 2)
    def _():
        acc_ref[...] = sblk
        stash_ref[0:D] = sb16

    @pl.when(i == 3)
    def _():
        acc_ref[...] = acc_ref[...] + sblk
        stash_ref[D:2 * D] = sb16

    @pl.when(i == 4)
    def _():
        acc_ref[...] = acc_ref[...] + sblk
        stash_ref[2 * D:3 * D] = sb16
        w1m_ref[...] = acc_ref[...].astype(BF)

    @pl.when(i == 5)
    def _():
        proj_ref[16:24] = jnp.dot(_pad_rows(msb_ref[...], 8).astype(BF), sb16,
                                  preferred_element_type=F32)

        def mlp_ln(st_ref, w1r, w2r, r0):
            hpre = jnp.dot(st_ref[...].astype(BF), w1r[...].astype(BF),
                           preferred_element_type=F32) + auxp_ref[r0:r0 + 1]
            hmid = hpre * jax.nn.sigmoid(hpre)
            hv = jnp.dot(hmid.astype(BF), w2r[...].astype(BF),
                         preferred_element_type=F32) + auxp_ref[r0 + 1:r0 + 2]
            return _ln_rows(hv, auxp_ref[r0 + 2:r0 + 3],
                            auxp_ref[r0 + 3:r0 + 4])

        mv = mlp_ln(mem_ref, mw1, mw2, 0)
        nv = mlp_ln(noi_ref, nw1, nw2, 4)
        rv = mlp_ln(res_ref, rw1, rw2, 8)
        catv = jnp.concatenate(
            [mv.astype(BF), nv.astype(BF), rv.astype(BF)], axis=-1)
        bias_ref[...] = (jnp.dot(catv, stash_ref[...],
                                 preferred_element_type=F32)
                         + auxp_ref[12:13]).reshape(bias_ref.shape)


def _tables_kernel(pwg_ref, pwb_ref, w2_ref, cpg_ref, tmg_ref, msg_ref,
                   pwcat_ref, w2b_ref, small_ref):
    D = pwg_ref.shape[1]
    npad = pwcat_ref.shape[0] - pwg_ref.shape[0]
    zp = jnp.zeros((npad, D), F32)
    pwcat_ref[:, 0:D] = jnp.concatenate(
        [pwg_ref[...] - 1.0, zp], axis=0).astype(BF)
    pwcat_ref[:, D:2 * D] = jnp.concatenate(
        [pwb_ref[...], zp], axis=0).astype(BF)
    w2b_ref[...] = w2_ref[...].astype(BF)
    small_ref[...] = jnp.concatenate(
        [_pad_rows(cpg_ref[...] - 1.0, 8), _pad_rows(tmg_ref[...] - 1.0, 8),
         _pad_rows(msg_ref[...] - 1.0, 8)], axis=0).astype(BF)


def _main_kernel(x_ref, ids_ref, pw_ref, small_ref, w1m_ref, proj_ref, w2_ref,
                 bias_ref, auxp_ref, o_ref):
    D = x_ref.shape[-1]
    tile = x_ref.shape[1]
    xb = x_ref[0]
    ids = ids_ref[0, 0]
    pid = ids[0:1, :]
    cid = ids[1:2, :]
    tid = ids[2:3, :]
    sid = ids[3:4, :]

    mu = jnp.mean(xb, axis=-1, keepdims=True)
    sxx = jnp.mean(xb * xb, axis=-1, keepdims=True)
    xhat = (xb - mu) * jax.lax.rsqrt(sxx - mu * mu + EPS)

    npw = pw_ref.shape[0]
    iota_pw = jax.lax.broadcasted_iota(jnp.int32, (npw, tile), 0)
    ohT_p = jnp.where(iota_pw == pid, 1.0, 0.0).astype(BF)
    pwgb = jax.lax.dot_general(ohT_p, pw_ref[...], _TA,
                               preferred_element_type=F32)
    h1 = xhat * (pwgb[:, :D] + 1.0) + pwgb[:, D:]

    mu1 = jnp.mean(h1, axis=-1, keepdims=True)
    s11 = jnp.mean(h1 * h1, axis=-1, keepdims=True)
    h1h = (h1 - mu1) * jax.lax.rsqrt(s11 - mu1 * mu1 + EPS)

    iota8 = jax.lax.broadcasted_iota(jnp.int32, (8, tile), 0)
    ohT_c = jnp.where(iota8 == cid, 1.0, 0.0).astype(BF)
    ohT_t = jnp.where(iota8 == tid, 1.0, 0.0).astype(BF)
    ohT_s = jnp.where(iota8 == sid, 1.0, 0.0).astype(BF)
    cg1 = jax.lax.dot_general(ohT_c, small_ref[0:8], _TA,
                              preferred_element_type=F32)
    tg1 = jax.lax.dot_general(ohT_t, small_ref[8:16], _TA,
                              preferred_element_type=F32)
    sg1 = jax.lax.dot_general(ohT_s, small_ref[16:24], _TA,
                              preferred_element_type=F32)

    h = h1h * (cg1 + 1.0)
    t = xhat * (tg1 + 1.0)
    s = xhat * (sg1 + 1.0)

    iota24 = jax.lax.broadcasted_iota(jnp.int32, (24, tile), 0)
    m24 = (iota24 == cid) | (iota24 - 8 == tid) | (iota24 - 16 == sid)
    oh24T = jnp.where(m24, 1.0, 0.0)
    badd = jax.lax.dot_general(oh24T, proj_ref[...], _TA,
                               preferred_element_type=F32)

    big = jnp.concatenate(
        [h.astype(BF), t.astype(BF), s.astype(BF), xb.astype(BF)], axis=-1)
    hD = D // 2
    parts = []
    for n in range(2):
        nsl = slice(n * hD, (n + 1) * hD)
        pre_n = (jnp.dot(big, w1m_ref[:, nsl], preferred_element_type=F32)
                 + badd[:, nsl] + bias_ref[0][:, nsl])
        hid_n = (pre_n * jax.nn.sigmoid(pre_n)).astype(BF)
        parts.append(jnp.dot(hid_n, w2_ref[nsl, :],
                             preferred_element_type=F32))
    h2 = parts[0] + parts[1] + auxp_ref[13:14]

    mu2 = jnp.mean(h2, axis=-1, keepdims=True)
    s22 = jnp.mean(h2 * h2, axis=-1, keepdims=True)
    o_ref[0] = ((h2 - mu2) * jax.lax.rsqrt(s22 - mu2 * mu2 + EPS)
                ) * auxp_ref[14:15] + auxp_ref[15:16]


def kernel(x, pathway_ids, compartment_ids, time_steps, scale_type,
           memory_state, noise_state, resource_state,
           pw_g, pw_b, cp_g, cp_b, tm_g, tm_b, ms_g, ms_b,
           mem_W1, mem_b1, mem_W2, mem_b2, mem_g, mem_be,
           noi_W1, noi_b1, noi_W2, noi_b2, noi_g, noi_be,
           res_W1, res_b1, res_W2, res_b2, res_g, res_be,
           int_W1, int_b1, int_W2, int_b2, int_g, int_be, aw):
    B, S, D = x.shape
    TILE = 512
    NB = S // TILE

    npw = pw_g.shape[0]
    npw_pad = ((npw + 127) // 128) * 128

    auxp = jnp.stack([mem_b1, mem_b2, mem_g, mem_be,
                      noi_b1, noi_b2, noi_g, noi_be,
                      res_b1, res_b2, res_g, res_be,
                      int_b1, int_b2, int_g, int_be], axis=0)

    def _w1m_row(i):
        return jnp.where(i < 2, i, jnp.where(i == 5, 2, 3))

    w1m, proj, bias_rows = pl.pallas_call(
        _weights_kernel,
        out_shape=(jax.ShapeDtypeStruct((4 * D, D), BF),
                   jax.ShapeDtypeStruct((24, D), F32),
                   jax.ShapeDtypeStruct((B, 1, D), F32)),
        grid=(6,),
        in_specs=[
            pl.BlockSpec((6,), lambda i: (0,)),
            pl.BlockSpec((D, D), lambda i: (i, 0)),
            pl.BlockSpec((B, 512), lambda i: (0, 0)),
            pl.BlockSpec((B, 64), lambda i: (0, 0)),
            pl.BlockSpec((B, 32), lambda i: (0, 0)),
            pl.BlockSpec((512, D), lambda i: (0, 0)),
            pl.BlockSpec((D, D), lambda i: (0, 0)),
            pl.BlockSpec((64, D), lambda i: (0, 0)),
            pl.BlockSpec((D, D), lambda i: (0, 0)),
            pl.BlockSpec((32, D), lambda i: (0, 0)),
            pl.BlockSpec((D, D), lambda i: (0, 0)),
            pl.BlockSpec((5, D), lambda i: (0, 0)),
            pl.BlockSpec((5, D), lambda i: (0, 0)),
            pl.BlockSpec((3, D), lambda i: (0, 0)),
            pl.BlockSpec((16, D), lambda i: (0, 0)),
        ],
        out_specs=(pl.BlockSpec((D, D), lambda i: (_w1m_row(i), 0)),
                   pl.BlockSpec((24, D), lambda i: (0, 0)),
                   pl.BlockSpec((B, 1, D), lambda i: (0, 0, 0))),
        scratch_shapes=[pltpu.VMEM((D, D), F32),
                        pltpu.VMEM((3 * D, D), BF)],
        compiler_params=_CompilerParams(
            dimension_semantics=("arbitrary",),
            vmem_limit_bytes=56 * 1024 * 1024,
        ),
        name="weights_prep",
    )(aw, int_W1, memory_state, noise_state, resource_state,
      mem_W1, mem_W2, noi_W1, noi_W2, res_W1, res_W2,
      cp_b, tm_b, ms_b, auxp)

    pw_cat, w2b, small = pl.pallas_call(
        _tables_kernel,
        out_shape=(jax.ShapeDtypeStruct((npw_pad, 2 * D), BF),
                   jax.ShapeDtypeStruct((D, D), BF),
                   jax.ShapeDtypeStruct((24, D), BF)),
        name="tables_prep",
    )(pw_g, pw_b, int_W2, cp_g, tm_g, ms_g)

    ids_all = jnp.stack(
        [pathway_ids.reshape(B, NB, TILE), compartment_ids.reshape(B, NB, TILE),
         time_steps.reshape(B, NB, TILE), scale_type.reshape(B, NB, TILE)],
        axis=2).astype(jnp.int32)

    out = pl.pallas_call(
        _main_kernel,
        out_shape=jax.ShapeDtypeStruct((B, S, D), F32),
        grid=(B, NB),
        in_specs=[
            pl.BlockSpec((1, TILE, D), lambda b, j: (b, j, 0)),
            pl.BlockSpec((1, 1, 4, TILE), lambda b, j: (b, j, 0, 0)),
            pl.BlockSpec((npw_pad, 2 * D), lambda b, j: (0, 0)),
            pl.BlockSpec((24, D), lambda b, j: (0, 0)),
            pl.BlockSpec((4 * D, D), lambda b, j: (0, 0)),
            pl.BlockSpec((24, D), lambda b, j: (0, 0)),
            pl.BlockSpec((D, D), lambda b, j: (0, 0)),
            pl.BlockSpec((1, 1, D), lambda b, j: (b, 0, 0)),
            pl.BlockSpec((16, D), lambda b, j: (0, 0)),
        ],
        out_specs=pl.BlockSpec((1, TILE, D), lambda b, j: (b, j, 0)),
        compiler_params=_CompilerParams(
            dimension_semantics=("parallel", "arbitrary"),
            vmem_limit_bytes=56 * 1024 * 1024,
        ),
        name="comprehensive_norm",
    )(x, ids_all, pw_cat, small, w1m, proj, w2b, bias_rows, auxp)
    return out


# speedup vs baseline: 1.0497x; 1.0063x over previous
import jax
import jax.numpy as jnp
from jax.experimental import pallas as pl
from jax.experimental.pallas import tpu as pltpu

EPS = 1e-5
BF = jnp.bfloat16
F32 = jnp.float32

_CompilerParams = getattr(pltpu, "CompilerParams", None) or pltpu.TPUCompilerParams

_TA = (((0,), (0,)), ((), ()))


def _ln_rows(h, g, be):
    m = jnp.mean(h, axis=-1, keepdims=True)
    c = h - m
    v = jnp.mean(c * c, axis=-1, keepdims=True)
    return c * jax.lax.rsqrt(v + EPS) * g + be


def _pad_rows(v, n):
    return jnp.concatenate(
        [v, jnp.zeros((n - v.shape[0], v.shape[1]), v.dtype)], axis=0)


def _weights_kernel(aw_ref, w1_ref, mem_ref, noi_ref, res_ref,
                    mw1, mw2, nw1, nw2, rw1, rw2,
                    cpb_ref, tmb_ref, msb_ref, auxp_ref,
                    w1m_ref, proj_ref, bias_ref,
                    acc_ref, stash_ref):
    i = pl.program_id(0)
    D = w1_ref.shape[0]
    av = aw_ref[...]
    ex = jnp.exp(av - jnp.max(av))
    wv = ex / jnp.sum(ex)
    wi = jnp.sum(jnp.where(jax.lax.iota(jnp.int32, 6) == i, wv, 0.0),
                 keepdims=True).reshape(1, 1)
    sblk = w1_ref[...] * wi
    sb16 = sblk.astype(BF)

    @pl.when((i == 0) | (i == 1) | (i == 5))
    def _():
        w1m_ref[...] = sb16

    @pl.when(i == 0)
    def _():
        proj_ref[0:8] = jnp.dot(_pad_rows(cpb_ref[...], 8).astype(BF), sb16,
                                preferred_element_type=F32)

    @pl.when(i == 1)
    def _():
        proj_ref[8:16] = jnp.dot(_pad_rows(tmb_ref[...], 8).astype(BF), sb16,
                                 preferred_element_type=F32)

    @pl.when(i == 2)
    def _():
        acc_ref[...] = sblk
        stash_ref[0:D] = sb16

    @pl.when(i == 3)
    def _():
        acc_ref[...] = acc_ref[...] + sblk
        stash_ref[D:2 * D] = sb16

    @pl.when(i == 4)
    def _():
        acc_ref[...] = acc_ref[...] + sblk
        stash_ref[2 * D:3 * D] = sb16
        w1m_ref[...] = acc_ref[...].astype(BF)

    @pl.when(i == 5)
    def _():
        proj_ref[16:24] = jnp.dot(_pad_rows(msb_ref[...], 8).astype(BF), sb16,
                                  preferred_element_type=F32)

        def mlp_ln(st_ref, w1r, w2r, r0):
            hpre = jnp.dot(st_ref[...].astype(BF), w1r[...].astype(BF),
                           preferred_element_type=F32) + auxp_ref[r0:r0 + 1]
            hmid = hpre * jax.nn.sigmoid(hpre)
            hv = jnp.dot(hmid.astype(BF), w2r[...].astype(BF),
                         preferred_element_type=F32) + auxp_ref[r0 + 1:r0 + 2]
            return _ln_rows(hv, auxp_ref[r0 + 2:r0 + 3],
                            auxp_ref[r0 + 3:r0 + 4])

        mv = mlp_ln(mem_ref, mw1, mw2, 0)
        nv = mlp_ln(noi_ref, nw1, nw2, 4)
        rv = mlp_ln(res_ref, rw1, rw2, 8)
        catv = jnp.concatenate(
            [mv.astype(BF), nv.astype(BF), rv.astype(BF)], axis=-1)
        bias_ref[...] = (jnp.dot(catv, stash_ref[...],
                                 preferred_element_type=F32)
                         + auxp_ref[12:13]).reshape(bias_ref.shape)


def _tables_kernel(pwg_ref, pwb_ref, w2_ref, cpg_ref, tmg_ref, msg_ref,
                   pwcat_ref, w2b_ref, small_ref):
    D = pwg_ref.shape[1]
    npad = pwcat_ref.shape[0] - pwg_ref.shape[0]
    zp = jnp.zeros((npad, D), F32)
    pwcat_ref[:, 0:D] = jnp.concatenate(
        [pwg_ref[...] - 1.0, zp], axis=0).astype(BF)
    pwcat_ref[:, D:2 * D] = jnp.concatenate(
        [pwb_ref[...], zp], axis=0).astype(BF)
    w2b_ref[...] = w2_ref[...].astype(BF)
    small_ref[...] = jnp.concatenate(
        [_pad_rows(cpg_ref[...] - 1.0, 8), _pad_rows(tmg_ref[...] - 1.0, 8),
         _pad_rows(msg_ref[...] - 1.0, 8)], axis=0).astype(BF)


def _main_kernel(x_ref, ids_ref, pw_ref, small_ref, w1m_ref, proj_ref, w2_ref,
                 bias_ref, auxp_ref, o_ref):
    D = x_ref.shape[-1]
    tile = x_ref.shape[1]
    xb = x_ref[0]
    ids = ids_ref[0, 0]
    pid = ids[0:1, :]
    cid = ids[1:2, :]
    tid = ids[2:3, :]
    sid = ids[3:4, :]

    mu = jnp.mean(xb, axis=-1, keepdims=True)
    sxx = jnp.mean(xb * xb, axis=-1, keepdims=True)
    xhat = (xb - mu) * jax.lax.rsqrt(sxx - mu * mu + EPS)

    npw = pw_ref.shape[0]
    iota_pw = jax.lax.broadcasted_iota(jnp.int32, (npw, tile), 0)
    ohT_p = jnp.where(iota_pw == pid, 1.0, 0.0).astype(BF)
    pwgb = jax.lax.dot_general(ohT_p, pw_ref[...], _TA,
                               preferred_element_type=F32)
    h1 = xhat * (pwgb[:, :D] + 1.0) + pwgb[:, D:]

    mu1 = jnp.mean(h1, axis=-1, keepdims=True)
    s11 = jnp.mean(h1 * h1, axis=-1, keepdims=True)
    h1h = (h1 - mu1) * jax.lax.rsqrt(s11 - mu1 * mu1 + EPS)

    iota8 = jax.lax.broadcasted_iota(jnp.int32, (8, tile), 0)
    ohT_c = jnp.where(iota8 == cid, 1.0, 0.0).astype(BF)
    ohT_t = jnp.where(iota8 == tid, 1.0, 0.0).astype(BF)
    ohT_s = jnp.where(iota8 == sid, 1.0, 0.0).astype(BF)
    cg1 = jax.lax.dot_general(ohT_c, small_ref[0:8], _TA,
                              preferred_element_type=F32)
    tg1 = jax.lax.dot_general(ohT_t, small_ref[8:16], _TA,
                              preferred_element_type=F32)
    sg1 = jax.lax.dot_general(ohT_s, small_ref[16:24], _TA,
                              preferred_element_type=F32)

    h = h1h * (cg1 + 1.0)
    t = xhat * (tg1 + 1.0)
    s = xhat * (sg1 + 1.0)

    iota24 = jax.lax.broadcasted_iota(jnp.int32, (24, tile), 0)
    m24 = (iota24 == cid) | (iota24 - 8 == tid) | (iota24 - 16 == sid)
    oh24T = jnp.where(m24, 1.0, 0.0)
    badd = jax.lax.dot_general(oh24T, proj_ref[...], _TA,
                               preferred_element_type=F32)

    big = jnp.concatenate(
        [h.astype(BF), t.astype(BF), s.astype(BF), xb.astype(BF)], axis=-1)
    pre = (jnp.dot(big, w1m_ref[...], preferred_element_type=F32)
           + badd + bias_ref[0])
    hid = pre * jax.nn.sigmoid(pre)
    h2 = jnp.dot(hid.astype(BF), w2_ref[...],
                 preferred_element_type=F32) + auxp_ref[13:14]

    mu2 = jnp.mean(h2, axis=-1, keepdims=True)
    s22 = jnp.mean(h2 * h2, axis=-1, keepdims=True)
    o_ref[0] = ((h2 - mu2) * jax.lax.rsqrt(s22 - mu2 * mu2 + EPS)
                ) * auxp_ref[14:15] + auxp_ref[15:16]


def kernel(x, pathway_ids, compartment_ids, time_steps, scale_type,
           memory_state, noise_state, resource_state,
           pw_g, pw_b, cp_g, cp_b, tm_g, tm_b, ms_g, ms_b,
           mem_W1, mem_b1, mem_W2, mem_b2, mem_g, mem_be,
           noi_W1, noi_b1, noi_W2, noi_b2, noi_g, noi_be,
           res_W1, res_b1, res_W2, res_b2, res_g, res_be,
           int_W1, int_b1, int_W2, int_b2, int_g, int_be, aw):
    B, S, D = x.shape
    TILE = 512
    NB = S // TILE

    npw = pw_g.shape[0]
    npw_pad = ((npw + 127) // 128) * 128

    auxp = jnp.stack([mem_b1, mem_b2, mem_g, mem_be,
                      noi_b1, noi_b2, noi_g, noi_be,
                      res_b1, res_b2, res_g, res_be,
                      int_b1, int_b2, int_g, int_be], axis=0)

    def _w1m_row(i):
        return jnp.where(i < 2, i, jnp.where(i == 5, 2, 3))

    w1m, proj, bias_rows = pl.pallas_call(
        _weights_kernel,
        out_shape=(jax.ShapeDtypeStruct((4 * D, D), BF),
                   jax.ShapeDtypeStruct((24, D), F32),
                   jax.ShapeDtypeStruct((B, 1, D), F32)),
        grid=(6,),
        in_specs=[
            pl.BlockSpec((6,), lambda i: (0,)),
            pl.BlockSpec((D, D), lambda i: (i, 0)),
            pl.BlockSpec((B, 512), lambda i: (0, 0)),
            pl.BlockSpec((B, 64), lambda i: (0, 0)),
            pl.BlockSpec((B, 32), lambda i: (0, 0)),
            pl.BlockSpec((512, D), lambda i: (0, 0)),
            pl.BlockSpec((D, D), lambda i: (0, 0)),
            pl.BlockSpec((64, D), lambda i: (0, 0)),
            pl.BlockSpec((D, D), lambda i: (0, 0)),
            pl.BlockSpec((32, D), lambda i: (0, 0)),
            pl.BlockSpec((D, D), lambda i: (0, 0)),
            pl.BlockSpec((5, D), lambda i: (0, 0)),
            pl.BlockSpec((5, D), lambda i: (0, 0)),
            pl.BlockSpec((3, D), lambda i: (0, 0)),
            pl.BlockSpec((16, D), lambda i: (0, 0)),
        ],
        out_specs=(pl.BlockSpec((D, D), lambda i: (_w1m_row(i), 0)),
                   pl.BlockSpec((24, D), lambda i: (0, 0)),
                   pl.BlockSpec((B, 1, D), lambda i: (0, 0, 0))),
        scratch_shapes=[pltpu.VMEM((D, D), F32),
                        pltpu.VMEM((3 * D, D), BF)],
        compiler_params=_CompilerParams(
            dimension_semantics=("arbitrary",),
            vmem_limit_bytes=56 * 1024 * 1024,
        ),
        name="weights_prep",
    )(aw, int_W1, memory_state, noise_state, resource_state,
      mem_W1, mem_W2, noi_W1, noi_W2, res_W1, res_W2,
      cp_b, tm_b, ms_b, auxp)

    pw_cat, w2b, small = pl.pallas_call(
        _tables_kernel,
        out_shape=(jax.ShapeDtypeStruct((npw_pad, 2 * D), BF),
                   jax.ShapeDtypeStruct((D, D), BF),
                   jax.ShapeDtypeStruct((24, D), BF)),
        name="tables_prep",
    )(pw_g, pw_b, int_W2, cp_g, tm_g, ms_g)

    ids_all = jnp.stack(
        [pathway_ids.reshape(B, NB, TILE), compartment_ids.reshape(B, NB, TILE),
         time_steps.reshape(B, NB, TILE), scale_type.reshape(B, NB, TILE)],
        axis=2).astype(jnp.int32)

    out = pl.pallas_call(
        _main_kernel,
        out_shape=jax.ShapeDtypeStruct((B, S, D), F32),
        grid=(B, NB),
        in_specs=[
            pl.BlockSpec((1, TILE, D), lambda b, j: (b, j, 0)),
            pl.BlockSpec((1, 1, 4, TILE), lambda b, j: (b, j, 0, 0)),
            pl.BlockSpec((npw_pad, 2 * D), lambda b, j: (0, 0)),
            pl.BlockSpec((24, D), lambda b, j: (0, 0)),
            pl.BlockSpec((4 * D, D), lambda b, j: (0, 0)),
            pl.BlockSpec((24, D), lambda b, j: (0, 0)),
            pl.BlockSpec((D, D), lambda b, j: (0, 0)),
            pl.BlockSpec((1, 1, D), lambda b, j: (b, 0, 0)),
            pl.BlockSpec((16, D), lambda b, j: (0, 0)),
        ],
        out_specs=pl.BlockSpec((1, TILE, D), lambda b, j: (b, j, 0)),
        compiler_params=_CompilerParams(
            dimension_semantics=("parallel", "arbitrary"),
            vmem_limit_bytes=56 * 1024 * 1024,
        ),
        name="comprehensive_norm",
    )(x, ids_all, pw_cat, small, w1m, proj, w2b, bias_rows, auxp)
    return out
